# Initial kernel scaffold; baseline (speedup 1.0000x reference)
import jax
import jax.numpy as jnp
from jax.experimental import pallas as pl


def kernel(x, w1_oihw, b1, w1_mat, w2_oihw, b2, w3_oihw, b3, wp_oihw, bp, wp_taps):
    raise NotImplementedError("write your pallas kernel here")


# speedup vs baseline: 49.1528x; 49.1528x over previous
import functools
import math

import numpy as np
import jax
import jax.numpy as jnp
from jax import lax
from jax.experimental import pallas as pl
from jax.experimental.pallas import tpu as pltpu


def _out_size(n, k, s, p):
    return (n + 2 * p - k) // s + 1


def _np_pool_matrix(H, W, k, s, p):
    Ho, Wo = _out_size(H, k, s, p), _out_size(W, k, s, p)
    P = np.zeros((H * W, Ho * Wo), np.float32)
    for qy in range(Ho):
        for qx in range(Wo):
            y0, x0 = qy * s - p, qx * s - p
            for dy in range(k):
                for dx in range(k):
                    yy, xx = y0 + dy, x0 + dx
                    if 0 <= yy < H and 0 <= xx < W:
                        P[yy * W + xx, qy * Wo + qx] = 1.0
    return P


def _np_conv_gather(Hi, Wi, Ho, Wo, stride):
    S = np.zeros((9, Hi * Wi, Ho * Wo), np.float32)
    for kk in range(9):
        dy, dx = kk // 3 - 1, kk % 3 - 1
        for qy in range(Ho):
            for qx in range(Wo):
                yy, xx = qy * stride + dy, qx * stride + dx
                if 0 <= yy < Hi and 0 <= xx < Wi:
                    S[kk, yy * Wi + xx, qy * Wo + qx] = 1.0
    return S


def _np_bilinear_matrix(Hs, Ws, H, W):
    def axis_weights(out_size, in_size):
        A = np.zeros((out_size, in_size), np.float64)
        scale = in_size / out_size
        for o in range(out_size):
            src = min(max((o + 0.5) * scale - 0.5, 0.0), in_size - 1)
            i0 = int(np.floor(src))
            i1 = min(i0 + 1, in_size - 1)
            frac = src - i0
            A[o, i0] += 1.0 - frac
            A[o, i1] += frac
        return A
    Ah = axis_weights(H, Hs)
    Aw = axis_weights(W, Ws)
    M = np.einsum("hi,wj->ijhw", Ah, Aw).reshape(Hs * Ws, H * W)
    return M.astype(np.float32)


def _np_tap_masks(H, W):
    p = np.arange(H * W)
    hh, ww = p // W, p % W
    masks = np.zeros((9, H * W), np.float32)
    for k in range(9):
        oy, ox = k // 3 - 1, k % 3 - 1
        valid = ((hh + oy >= 0) & (hh + oy < H)
                 & (ww + ox >= 0) & (ww + ox < W))
        masks[k] = valid.astype(np.float32)
    return masks


def _np_gate_select(TB, C):
    sel = np.zeros((TB * C, TB * C), np.float32)
    rows = np.arange(TB * C)
    sel[rows, (rows // C) * C] = 1.0
    return sel


def _fused_kernel(x_ref, w1_ref, b1_ref, pool_ref, w2_ref, b2_ref, s2_ref,
                  w3_ref, b3_ref, s3_ref, up_ref, gsel_ref, wp_ref, bp_ref,
                  m_ref, o_ref, *, W, HW):
    f32 = jnp.float32
    x = x_ref[...]

    x1 = jnp.dot(w1_ref[...], x, preferred_element_type=f32) + b1_ref[...]

    mx = jnp.max(x1, axis=1, keepdims=True)
    e = jnp.exp(x1 - mx)
    num = jnp.dot(e * x1, pool_ref[...], preferred_element_type=f32)
    den = jnp.dot(e, pool_ref[...], preferred_element_type=f32)
    x2 = num / den

    a2 = jnp.broadcast_to(b2_ref[...], (b2_ref.shape[0], s2_ref.shape[2]))
    for k in range(9):
        g = jnp.dot(x2, s2_ref[k], preferred_element_type=f32)
        a2 = a2 + jnp.dot(w2_ref[k], g, preferred_element_type=f32)

    a3 = jnp.broadcast_to(b3_ref[...], (b3_ref.shape[0], s3_ref.shape[2]))
    for k in range(9):
        g = jnp.dot(a2, s3_ref[k], preferred_element_type=f32)
        a3 = a3 + jnp.dot(w3_ref[k], g, preferred_element_type=f32)
    ys = jax.nn.sigmoid(a3)

    y = jnp.dot(ys, up_ref[...], preferred_element_type=f32)
    z = x * y
    g2 = jax.nn.sigmoid(jnp.dot(gsel_ref[...], x, preferred_element_type=f32))

    acc = jnp.dot(wp_ref[4], z, preferred_element_type=f32)
    for k in range(9):
        if k == 4:
            continue
        s = (k // 3 - 1) * W + (k % 3 - 1)
        tap = pltpu.roll(z, shift=(-s) % HW, axis=1) * m_ref[k:k + 1, :]
        acc = acc + jnp.dot(wp_ref[k], tap, preferred_element_type=f32)

    o_ref[...] = z * g2 + acc + bp_ref[...]


def _forward(x, params):
    orig_shape = x.shape
    if x.ndim == 5:
        n, s, c, h, w = x.shape
        x4d = x.reshape(n * s, c, h, w)
    elif x.ndim == 4:
        x4d = x
    else:
        raise ValueError("Input tensor must be 4D or 5D")
    B, C, H, W = x4d.shape
    HW = H * W
    f = params["b1"].shape[0]

    TB = 1
    for cand in (16, 8, 4, 2):
        if B % cand == 0:
            TB = cand
            break
    grid = B // TB

    Hp, Wp = _out_size(H, 7, 3, 1), _out_size(W, 7, 3, 1)
    H2, W2 = _out_size(Hp, 3, 2, 1), _out_size(Wp, 3, 2, 1)
    n_pool, n2 = Hp * Wp, H2 * W2

    x2d = x4d.reshape(B * C, HW)

    eye = jnp.eye(TB, dtype=jnp.float32)
    w1_bd = jnp.einsum("bd,oc->bodc", eye,
                       params["w1_mat"]).reshape(TB * f, TB * C)
    b1_t = jnp.tile(params["b1"], TB).reshape(TB * f, 1)
    w2_taps = jnp.transpose(params["w2_oihw"], (2, 3, 0, 1)).reshape(9, f, f)
    w2_bd = jnp.einsum("bd,koi->kbodi", eye,
                       w2_taps).reshape(9, TB * f, TB * f)
    b2_t = jnp.tile(params["b2"], TB).reshape(TB * f, 1)
    w3_taps = jnp.transpose(params["w3_oihw"], (2, 3, 0, 1)).reshape(9, C, f)
    w3_bd = jnp.einsum("bd,koi->kbodi", eye,
                       w3_taps).reshape(9, TB * C, TB * f)
    b3_t = jnp.tile(params["b3"], TB).reshape(TB * C, 1)
    wp_bd = jnp.einsum("bd,koi->kbodi", eye,
                       params["wp_taps"]).reshape(9, TB * C, TB * C)
    bp_t = jnp.tile(params["bp"], TB).reshape(TB * C, 1)

    pool_mat = jnp.asarray(_np_pool_matrix(H, W, 7, 3, 1))
    s2 = jnp.asarray(_np_conv_gather(Hp, Wp, H2, W2, 2))
    s3 = jnp.asarray(_np_conv_gather(H2, W2, H2, W2, 1))
    up_mat = jnp.asarray(_np_bilinear_matrix(H2, W2, H, W))
    masks = jnp.asarray(_np_tap_masks(H, W))
    gsel = jnp.asarray(_np_gate_select(TB, C))

    body = functools.partial(_fused_kernel, W=W, HW=HW)
    out2d = pl.pallas_call(
        body,
        out_shape=jax.ShapeDtypeStruct((B * C, HW), jnp.float32),
        grid=(grid,),
        in_specs=[
            pl.BlockSpec((TB * C, HW), lambda i: (i, 0)),
            pl.BlockSpec((TB * f, TB * C), lambda i: (0, 0)),
            pl.BlockSpec((TB * f, 1), lambda i: (0, 0)),
            pl.BlockSpec((HW, n_pool), lambda i: (0, 0)),
            pl.BlockSpec((9, TB * f, TB * f), lambda i: (0, 0, 0)),
            pl.BlockSpec((TB * f, 1), lambda i: (0, 0)),
            pl.BlockSpec((9, n_pool, n2), lambda i: (0, 0, 0)),
            pl.BlockSpec((9, TB * C, TB * f), lambda i: (0, 0, 0)),
            pl.BlockSpec((TB * C, 1), lambda i: (0, 0)),
            pl.BlockSpec((9, n2, n2), lambda i: (0, 0, 0)),
            pl.BlockSpec((n2, HW), lambda i: (0, 0)),
            pl.BlockSpec((TB * C, TB * C), lambda i: (0, 0)),
            pl.BlockSpec((9, TB * C, TB * C), lambda i: (0, 0, 0)),
            pl.BlockSpec((TB * C, 1), lambda i: (0, 0)),
            pl.BlockSpec((9, HW), lambda i: (0, 0)),
        ],
        out_specs=pl.BlockSpec((TB * C, HW), lambda i: (i, 0)),
        compiler_params=pltpu.CompilerParams(
            dimension_semantics=("parallel",),
            vmem_limit_bytes=64 << 20),
        cost_estimate=pl.CostEstimate(
            flops=2 * B * HW * (C * f + 2 * f * n_pool // 5 + 10 * C * C)
            + 2 * B * f * 9 * (f * n2 + n_pool * n2 // 4),
            transcendentals=B * (f + C) * HW,
            bytes_accessed=4 * 2 * B * C * HW),
    )(x2d, w1_bd, b1_t, pool_mat, w2_bd, b2_t, s2, w3_bd, b3_t, s3,
      up_mat, gsel, wp_bd, bp_t, masks)

    out = out2d.reshape(B, C, H, W)
    return out.reshape(orig_shape)


def kernel(x, w1_oihw, b1, w1_mat, w2_oihw, b2, w3_oihw, b3, wp_oihw, bp,
           wp_taps):
    params = dict(
        w1_mat=w1_mat, b1=b1,
        w2_oihw=w2_oihw, b2=b2,
        w3_oihw=w3_oihw, b3=b3,
        wp_taps=wp_taps, bp=bp,
    )
    return _forward(x, params)


# speedup vs baseline: 67.6841x; 1.3770x over previous
import functools
import math

import numpy as np
import jax
import jax.numpy as jnp
from jax import lax
from jax.experimental import pallas as pl
from jax.experimental.pallas import tpu as pltpu

_BF = jnp.bfloat16


def _out_size(n, k, s, p):
    return (n + 2 * p - k) // s + 1


def _np_pool_matrix(H, W, k, s, p):
    Ho, Wo = _out_size(H, k, s, p), _out_size(W, k, s, p)
    P = np.zeros((H * W, Ho * Wo), np.float32)
    for qy in range(Ho):
        for qx in range(Wo):
            y0, x0 = qy * s - p, qx * s - p
            for dy in range(k):
                for dx in range(k):
                    yy, xx = y0 + dy, x0 + dx
                    if 0 <= yy < H and 0 <= xx < W:
                        P[yy * W + xx, qy * Wo + qx] = 1.0
    return P


def _np_conv_gather(Hi, Wi, Ho, Wo, stride):
    S = np.zeros((9, Hi * Wi, Ho * Wo), np.float32)
    for kk in range(9):
        dy, dx = kk // 3 - 1, kk % 3 - 1
        for qy in range(Ho):
            for qx in range(Wo):
                yy, xx = qy * stride + dy, qx * stride + dx
                if 0 <= yy < Hi and 0 <= xx < Wi:
                    S[kk, yy * Wi + xx, qy * Wo + qx] = 1.0
    return S


def _np_bilinear_matrix(Hs, Ws, H, W):
    def axis_weights(out_size, in_size):
        A = np.zeros((out_size, in_size), np.float64)
        scale = in_size / out_size
        for o in range(out_size):
            src = min(max((o + 0.5) * scale - 0.5, 0.0), in_size - 1)
            i0 = int(np.floor(src))
            i1 = min(i0 + 1, in_size - 1)
            frac = src - i0
            A[o, i0] += 1.0 - frac
            A[o, i1] += frac
        return A
    Ah = axis_weights(H, Hs)
    Aw = axis_weights(W, Ws)
    M = np.einsum("hi,wj->ijhw", Ah, Aw).reshape(Hs * Ws, H * W)
    return M.astype(np.float32)


def _np_tap_masks(H, W):
    p = np.arange(H * W)
    hh, ww = p // W, p % W
    masks = np.zeros((9, H * W), np.float32)
    for k in range(9):
        oy, ox = k // 3 - 1, k % 3 - 1
        valid = ((hh + oy >= 0) & (hh + oy < H)
                 & (ww + ox >= 0) & (ww + ox < W))
        masks[k] = valid.astype(np.float32)
    return masks


def _np_gate_select(TB, C):
    sel = np.zeros((TB * C, TB * C), np.float32)
    rows = np.arange(TB * C)
    sel[rows, (rows // C) * C] = 1.0
    return sel


def _fused_kernel(x_ref, w1_ref, b1_ref, pool_ref, w2_ref, b2_ref, s2_ref,
                  w3_ref, b3_ref, s3_ref, up_ref, gsel_ref, wp_ref, bp_ref,
                  m_ref, o_ref, *, W, HW, halves):
    rows = x_ref.shape[0] // halves
    for h in range(halves):
        _one_group(x_ref, w1_ref, b1_ref, pool_ref, w2_ref, b2_ref, s2_ref,
                   w3_ref, b3_ref, s3_ref, up_ref, gsel_ref, wp_ref, bp_ref,
                   m_ref, o_ref, W=W, HW=HW, r0=h * rows, r1=(h + 1) * rows)


def _one_group(x_ref, w1_ref, b1_ref, pool_ref, w2_ref, b2_ref, s2_ref,
               w3_ref, b3_ref, s3_ref, up_ref, gsel_ref, wp_ref, bp_ref,
               m_ref, o_ref, *, W, HW, r0, r1):
    f32 = jnp.float32
    x = x_ref[r0:r1, :]
    xb = x.astype(_BF)

    x1 = (jnp.dot(w1_ref[...], xb, preferred_element_type=f32)
          + b1_ref[:, 0:1])

    mx = jnp.max(x1, axis=1, keepdims=True)
    e = jnp.exp(x1 - mx)
    num = jnp.dot((e * x1).astype(_BF), pool_ref[...],
                  preferred_element_type=f32)
    den = jnp.dot(e.astype(_BF), pool_ref[...], preferred_element_type=f32)
    x2 = (num / den).astype(_BF)

    n2 = s2_ref.shape[1] // 9
    g2all = jnp.dot(x2, s2_ref[...], preferred_element_type=f32).astype(_BF)
    gstack = jnp.concatenate(
        [g2all[:, k * n2:(k + 1) * n2] for k in range(9)], axis=0)
    a2 = (jnp.dot(w2_ref[...], gstack, preferred_element_type=f32)
          + b2_ref[:, 0:1])

    a2 = a2.astype(_BF)
    g3all = jnp.dot(a2, s3_ref[...], preferred_element_type=f32).astype(_BF)
    g3stack = jnp.concatenate(
        [g3all[:, k * n2:(k + 1) * n2] for k in range(9)], axis=0)
    a3 = (jnp.dot(w3_ref[...], g3stack, preferred_element_type=f32)
          + b3_ref[:, 0:1])
    ys = jax.nn.sigmoid(a3).astype(_BF)

    y = jnp.dot(ys, up_ref[...], preferred_element_type=f32)
    z = x * y
    zb = z.astype(_BF)
    g2 = jax.nn.sigmoid(jnp.dot(gsel_ref[...], xb, preferred_element_type=f32))

    taps = []
    for k in range(9):
        if k == 4:
            taps.append(zb)
            continue
        s = (k // 3 - 1) * W + (k % 3 - 1)
        taps.append(pltpu.roll(zb, shift=(-s) % HW, axis=1) * m_ref[k:k + 1, :])
    zstack = jnp.concatenate(taps, axis=0)
    acc = jnp.dot(wp_ref[...], zstack, preferred_element_type=f32)

    o_ref[r0:r1, :] = z * g2 + acc + bp_ref[:, 0:1]


def _forward(x, params):
    orig_shape = x.shape
    if x.ndim == 5:
        n, s, c, h, w = x.shape
        x4d = x.reshape(n * s, c, h, w)
    elif x.ndim == 4:
        x4d = x
    else:
        raise ValueError("Input tensor must be 4D or 5D")
    B, C, H, W = x4d.shape
    HW = H * W
    f = params["b1"].shape[0]

    TB = 1
    for cand in (16, 8, 4, 2):
        if B % cand == 0:
            TB = cand
            break
    halves = 2 if B % (2 * TB) == 0 else 1
    grid = B // (TB * halves)

    Hp, Wp = _out_size(H, 7, 3, 1), _out_size(W, 7, 3, 1)
    H2, W2 = _out_size(Hp, 3, 2, 1), _out_size(Wp, 3, 2, 1)
    n_pool, n2 = Hp * Wp, H2 * W2

    x2d = x4d.reshape(B * C, HW)

    eye = jnp.eye(TB, dtype=jnp.float32)
    w1_bd = jnp.einsum("bd,oc->bodc", eye,
                       params["w1_mat"]).reshape(TB * f, TB * C).astype(_BF)
    w2_taps = jnp.transpose(params["w2_oihw"], (2, 3, 0, 1)).reshape(9, f, f)
    w2_cat = jnp.einsum("bd,koi->bokdi", eye,
                        w2_taps).reshape(TB * f, 9 * TB * f).astype(_BF)
    w3_taps = jnp.transpose(params["w3_oihw"], (2, 3, 0, 1)).reshape(9, C, f)
    w3_cat = jnp.einsum("bd,koi->bokdi", eye,
                        w3_taps).reshape(TB * C, 9 * TB * f).astype(_BF)
    wp_cat = jnp.einsum("bd,koi->bokdi", eye,
                        params["wp_taps"]).reshape(TB * C, 9 * TB * C).astype(_BF)
    b1_t = jnp.broadcast_to(jnp.tile(params["b1"], TB)[:, None], (TB * f, 128))
    b2_t = jnp.broadcast_to(jnp.tile(params["b2"], TB)[:, None], (TB * f, 128))
    b3_t = jnp.broadcast_to(jnp.tile(params["b3"], TB)[:, None], (TB * C, 128))
    bp_t = jnp.broadcast_to(jnp.tile(params["bp"], TB)[:, None], (TB * C, 128))

    pool_mat = jnp.asarray(_np_pool_matrix(H, W, 7, 3, 1), dtype=_BF)
    s2_np = _np_conv_gather(Hp, Wp, H2, W2, 2)
    s2 = jnp.asarray(np.transpose(s2_np, (1, 0, 2)).reshape(n_pool, 9 * n2),
                     dtype=_BF)
    s3_np = _np_conv_gather(H2, W2, H2, W2, 1)
    s3 = jnp.asarray(np.transpose(s3_np, (1, 0, 2)).reshape(n2, 9 * n2),
                     dtype=_BF)
    up_mat = jnp.asarray(_np_bilinear_matrix(H2, W2, H, W), dtype=_BF)
    masks = jnp.asarray(_np_tap_masks(H, W), dtype=_BF)
    gsel = jnp.asarray(_np_gate_select(TB, C), dtype=_BF)

    body = functools.partial(_fused_kernel, W=W, HW=HW, halves=halves)
    out2d = pl.pallas_call(
        body,
        out_shape=jax.ShapeDtypeStruct((B * C, HW), jnp.float32),
        grid=(grid,),
        in_specs=[
            pl.BlockSpec((halves * TB * C, HW), lambda i: (i, 0)),
            pl.BlockSpec((TB * f, TB * C), lambda i: (0, 0)),
            pl.BlockSpec((TB * f, 128), lambda i: (0, 0)),
            pl.BlockSpec((HW, n_pool), lambda i: (0, 0)),
            pl.BlockSpec((TB * f, 9 * TB * f), lambda i: (0, 0)),
            pl.BlockSpec((TB * f, 128), lambda i: (0, 0)),
            pl.BlockSpec((n_pool, 9 * n2), lambda i: (0, 0)),
            pl.BlockSpec((TB * C, 9 * TB * f), lambda i: (0, 0)),
            pl.BlockSpec((TB * C, 128), lambda i: (0, 0)),
            pl.BlockSpec((n2, 9 * n2), lambda i: (0, 0)),
            pl.BlockSpec((n2, HW), lambda i: (0, 0)),
            pl.BlockSpec((TB * C, TB * C), lambda i: (0, 0)),
            pl.BlockSpec((TB * C, 9 * TB * C), lambda i: (0, 0)),
            pl.BlockSpec((TB * C, 128), lambda i: (0, 0)),
            pl.BlockSpec((9, HW), lambda i: (0, 0)),
        ],
        out_specs=pl.BlockSpec((halves * TB * C, HW), lambda i: (i, 0)),
        compiler_params=pltpu.CompilerParams(
            dimension_semantics=("parallel",),
            vmem_limit_bytes=64 << 20),
        cost_estimate=pl.CostEstimate(
            flops=2 * B * HW * (C * f + 2 * f * n_pool // 5 + 10 * C * C)
            + 2 * B * f * 9 * (f * n2 + n_pool * n2 // 4),
            transcendentals=B * (f + C) * HW,
            bytes_accessed=4 * 2 * B * C * HW),
    )(x2d, w1_bd, b1_t, pool_mat, w2_cat, b2_t, s2, w3_cat, b3_t, s3,
      up_mat, gsel, wp_cat, bp_t, masks)

    out = out2d.reshape(B, C, H, W)
    return out.reshape(orig_shape)


def kernel(x, w1_oihw, b1, w1_mat, w2_oihw, b2, w3_oihw, b3, wp_oihw, bp,
           wp_taps):
    params = dict(
        w1_mat=w1_mat, b1=b1,
        w2_oihw=w2_oihw, b2=b2,
        w3_oihw=w3_oihw, b3=b3,
        wp_taps=wp_taps, bp=bp,
    )
    return _forward(x, params)


# speedup vs baseline: 68.3675x; 1.0101x over previous
import functools
import math

import numpy as np
import jax
import jax.numpy as jnp
from jax import lax
from jax.experimental import pallas as pl
from jax.experimental.pallas import tpu as pltpu

_BF = jnp.bfloat16


def _out_size(n, k, s, p):
    return (n + 2 * p - k) // s + 1


def _np_pool_matrix(H, W, k, s, p):
    Ho, Wo = _out_size(H, k, s, p), _out_size(W, k, s, p)
    P = np.zeros((H * W, Ho * Wo), np.float32)
    for qy in range(Ho):
        for qx in range(Wo):
            y0, x0 = qy * s - p, qx * s - p
            for dy in range(k):
                for dx in range(k):
                    yy, xx = y0 + dy, x0 + dx
                    if 0 <= yy < H and 0 <= xx < W:
                        P[yy * W + xx, qy * Wo + qx] = 1.0
    return P


def _np_conv_gather(Hi, Wi, Ho, Wo, stride):
    S = np.zeros((9, Hi * Wi, Ho * Wo), np.float32)
    for kk in range(9):
        dy, dx = kk // 3 - 1, kk % 3 - 1
        for qy in range(Ho):
            for qx in range(Wo):
                yy, xx = qy * stride + dy, qx * stride + dx
                if 0 <= yy < Hi and 0 <= xx < Wi:
                    S[kk, yy * Wi + xx, qy * Wo + qx] = 1.0
    return S


def _np_bilinear_matrix(Hs, Ws, H, W):
    def axis_weights(out_size, in_size):
        A = np.zeros((out_size, in_size), np.float64)
        scale = in_size / out_size
        for o in range(out_size):
            src = min(max((o + 0.5) * scale - 0.5, 0.0), in_size - 1)
            i0 = int(np.floor(src))
            i1 = min(i0 + 1, in_size - 1)
            frac = src - i0
            A[o, i0] += 1.0 - frac
            A[o, i1] += frac
        return A
    Ah = axis_weights(H, Hs)
    Aw = axis_weights(W, Ws)
    M = np.einsum("hi,wj->ijhw", Ah, Aw).reshape(Hs * Ws, H * W)
    return M.astype(np.float32)


def _np_tap_masks(H, W):
    p = np.arange(H * W)
    hh, ww = p // W, p % W
    masks = np.zeros((9, H * W), np.float32)
    for k in range(9):
        oy, ox = k // 3 - 1, k % 3 - 1
        valid = ((hh + oy >= 0) & (hh + oy < H)
                 & (ww + ox >= 0) & (ww + ox < W))
        masks[k] = valid.astype(np.float32)
    return masks


def _np_gate_select(TB, C):
    sel = np.zeros((TB * C, TB * C), np.float32)
    rows = np.arange(TB * C)
    sel[rows, (rows // C) * C] = 1.0
    return sel


def _fused_kernel(x_ref, w1_ref, b1_ref, pool_ref, w2_ref, b2_ref, s2_ref,
                  w3_ref, b3_ref, s3_ref, up_ref, gsel_ref, wp_ref, bp_ref,
                  m_ref, o_ref, *, W, HW, halves):
    rows = x_ref.shape[0] // halves
    for h in range(halves):
        _one_group(x_ref, w1_ref, b1_ref, pool_ref, w2_ref, b2_ref, s2_ref,
                   w3_ref, b3_ref, s3_ref, up_ref, gsel_ref, wp_ref, bp_ref,
                   m_ref, o_ref, W=W, HW=HW, r0=h * rows, r1=(h + 1) * rows)


def _one_group(x_ref, w1_ref, b1_ref, pool_ref, w2_ref, b2_ref, s2_ref,
               w3_ref, b3_ref, s3_ref, up_ref, gsel_ref, wp_ref, bp_ref,
               m_ref, o_ref, *, W, HW, r0, r1):
    f32 = jnp.float32
    x = x_ref[r0:r1, :]
    xb = x.astype(_BF)

    x1 = (jnp.dot(w1_ref[...], xb, preferred_element_type=f32)
          + b1_ref[:, 0:1])

    mx = jnp.max(x1, axis=1, keepdims=True)
    e = jnp.exp(x1 - mx)
    num = jnp.dot((e * x1).astype(_BF), pool_ref[...],
                  preferred_element_type=f32)
    den = jnp.dot(e.astype(_BF), pool_ref[...], preferred_element_type=f32)
    x2 = (num / den).astype(_BF)

    n2 = s2_ref.shape[1] // 9
    g2all = jnp.dot(x2, s2_ref[...], preferred_element_type=f32).astype(_BF)
    gstack = jnp.concatenate(
        [g2all[:, k * n2:(k + 1) * n2] for k in range(9)], axis=0)
    a2 = (jnp.dot(w2_ref[...], gstack, preferred_element_type=f32)
          + b2_ref[:, 0:1])

    a2 = a2.astype(_BF)
    g3all = jnp.dot(a2, s3_ref[...], preferred_element_type=f32).astype(_BF)
    g3stack = jnp.concatenate(
        [g3all[:, k * n2:(k + 1) * n2] for k in range(9)], axis=0)
    a3 = (jnp.dot(w3_ref[...], g3stack, preferred_element_type=f32)
          + b3_ref[:, 0:1])
    ys = jax.nn.sigmoid(a3).astype(_BF)

    y = jnp.dot(ys, up_ref[...], preferred_element_type=f32)
    z = x * y
    zb = z.astype(_BF)
    g2 = jax.nn.sigmoid(jnp.dot(gsel_ref[...], xb, preferred_element_type=f32))

    taps = []
    for k in range(9):
        if k == 4:
            taps.append(zb)
            continue
        s = (k // 3 - 1) * W + (k % 3 - 1)
        taps.append(pltpu.roll(zb, shift=(-s) % HW, axis=1) * m_ref[k:k + 1, :])
    zstack = jnp.concatenate(taps, axis=0)
    acc = jnp.dot(wp_ref[...], zstack, preferred_element_type=f32)

    o_ref[r0:r1, :] = z * g2 + acc + bp_ref[:, 0:1]


def _forward(x, params):
    orig_shape = x.shape
    if x.ndim == 5:
        n, s, c, h, w = x.shape
        x4d = x.reshape(n * s, c, h, w)
    elif x.ndim == 4:
        x4d = x
    else:
        raise ValueError("Input tensor must be 4D or 5D")
    B, C, H, W = x4d.shape
    HW = H * W
    f = params["b1"].shape[0]

    TB = 1
    for cand in (16, 8, 4, 2):
        if B % cand == 0:
            TB = cand
            break
    halves = 2 if B % (2 * TB) == 0 else 1
    grid = B // (TB * halves)

    Hp, Wp = _out_size(H, 7, 3, 1), _out_size(W, 7, 3, 1)
    H2, W2 = _out_size(Hp, 3, 2, 1), _out_size(Wp, 3, 2, 1)
    n_pool, n2 = Hp * Wp, H2 * W2

    x2d = x4d.reshape(B * C, HW)

    eye = jnp.eye(TB, dtype=jnp.float32)
    w1_bd = jnp.einsum("bd,oc->bodc", eye,
                       params["w1_mat"]).reshape(TB * f, TB * C).astype(_BF)
    w2_cat = jnp.einsum("bd,oiyx->boyxdi", eye,
                        params["w2_oihw"]).reshape(TB * f, 9 * TB * f).astype(_BF)
    w3_cat = jnp.einsum("bd,oiyx->boyxdi", eye,
                        params["w3_oihw"]).reshape(TB * C, 9 * TB * f).astype(_BF)
    wp_cat = jnp.einsum("bd,koi->bokdi", eye,
                        params["wp_taps"]).reshape(TB * C, 9 * TB * C).astype(_BF)
    b1_t = jnp.broadcast_to(jnp.tile(params["b1"], TB)[:, None], (TB * f, 128))
    b2_t = jnp.broadcast_to(jnp.tile(params["b2"], TB)[:, None], (TB * f, 128))
    b3_t = jnp.broadcast_to(jnp.tile(params["b3"], TB)[:, None], (TB * C, 128))
    bp_t = jnp.broadcast_to(jnp.tile(params["bp"], TB)[:, None], (TB * C, 128))

    pool_mat = jnp.asarray(_np_pool_matrix(H, W, 7, 3, 1), dtype=_BF)
    s2_np = _np_conv_gather(Hp, Wp, H2, W2, 2)
    s2 = jnp.asarray(np.transpose(s2_np, (1, 0, 2)).reshape(n_pool, 9 * n2),
                     dtype=_BF)
    s3_np = _np_conv_gather(H2, W2, H2, W2, 1)
    s3 = jnp.asarray(np.transpose(s3_np, (1, 0, 2)).reshape(n2, 9 * n2),
                     dtype=_BF)
    up_mat = jnp.asarray(_np_bilinear_matrix(H2, W2, H, W), dtype=_BF)
    masks = jnp.asarray(_np_tap_masks(H, W), dtype=_BF)
    gsel = jnp.asarray(_np_gate_select(TB, C), dtype=_BF)

    body = functools.partial(_fused_kernel, W=W, HW=HW, halves=halves)
    out2d = pl.pallas_call(
        body,
        out_shape=jax.ShapeDtypeStruct((B * C, HW), jnp.float32),
        grid=(grid,),
        in_specs=[
            pl.BlockSpec((halves * TB * C, HW), lambda i: (i, 0)),
            pl.BlockSpec((TB * f, TB * C), lambda i: (0, 0)),
            pl.BlockSpec((TB * f, 128), lambda i: (0, 0)),
            pl.BlockSpec((HW, n_pool), lambda i: (0, 0)),
            pl.BlockSpec((TB * f, 9 * TB * f), lambda i: (0, 0)),
            pl.BlockSpec((TB * f, 128), lambda i: (0, 0)),
            pl.BlockSpec((n_pool, 9 * n2), lambda i: (0, 0)),
            pl.BlockSpec((TB * C, 9 * TB * f), lambda i: (0, 0)),
            pl.BlockSpec((TB * C, 128), lambda i: (0, 0)),
            pl.BlockSpec((n2, 9 * n2), lambda i: (0, 0)),
            pl.BlockSpec((n2, HW), lambda i: (0, 0)),
            pl.BlockSpec((TB * C, TB * C), lambda i: (0, 0)),
            pl.BlockSpec((TB * C, 9 * TB * C), lambda i: (0, 0)),
            pl.BlockSpec((TB * C, 128), lambda i: (0, 0)),
            pl.BlockSpec((9, HW), lambda i: (0, 0)),
        ],
        out_specs=pl.BlockSpec((halves * TB * C, HW), lambda i: (i, 0)),
        compiler_params=pltpu.CompilerParams(
            dimension_semantics=("parallel",),
            allow_input_fusion=[False] + [True] * 14,
            vmem_limit_bytes=64 << 20),
        cost_estimate=pl.CostEstimate(
            flops=2 * B * HW * (C * f + 2 * f * n_pool // 5 + 10 * C * C)
            + 2 * B * f * 9 * (f * n2 + n_pool * n2 // 4),
            transcendentals=B * (f + C) * HW,
            bytes_accessed=4 * 2 * B * C * HW),
    )(x2d, w1_bd, b1_t, pool_mat, w2_cat, b2_t, s2, w3_cat, b3_t, s3,
      up_mat, gsel, wp_cat, bp_t, masks)

    out = out2d.reshape(B, C, H, W)
    return out.reshape(orig_shape)


def kernel(x, w1_oihw, b1, w1_mat, w2_oihw, b2, w3_oihw, b3, wp_oihw, bp,
           wp_taps):
    params = dict(
        w1_mat=w1_mat, b1=b1,
        w2_oihw=w2_oihw, b2=b2,
        w3_oihw=w3_oihw, b3=b3,
        wp_taps=wp_taps, bp=bp,
    )
    return _forward(x, params)


# speedup vs baseline: 68.6930x; 1.0048x over previous
import functools
import math

import numpy as np
import jax
import jax.numpy as jnp
from jax import lax
from jax.experimental import pallas as pl
from jax.experimental.pallas import tpu as pltpu

_BF = jnp.bfloat16


def _out_size(n, k, s, p):
    return (n + 2 * p - k) // s + 1


def _np_pool_matrix(H, W, k, s, p):
    Ho, Wo = _out_size(H, k, s, p), _out_size(W, k, s, p)
    P = np.zeros((H * W, Ho * Wo), np.float32)
    for qy in range(Ho):
        for qx in range(Wo):
            y0, x0 = qy * s - p, qx * s - p
            for dy in range(k):
                for dx in range(k):
                    yy, xx = y0 + dy, x0 + dx
                    if 0 <= yy < H and 0 <= xx < W:
                        P[yy * W + xx, qy * Wo + qx] = 1.0
    return P


def _np_conv_gather(Hi, Wi, Ho, Wo, stride):
    S = np.zeros((9, Hi * Wi, Ho * Wo), np.float32)
    for kk in range(9):
        dy, dx = kk // 3 - 1, kk % 3 - 1
        for qy in range(Ho):
            for qx in range(Wo):
                yy, xx = qy * stride + dy, qx * stride + dx
                if 0 <= yy < Hi and 0 <= xx < Wi:
                    S[kk, yy * Wi + xx, qy * Wo + qx] = 1.0
    return S


def _np_bilinear_matrix(Hs, Ws, H, W):
    def axis_weights(out_size, in_size):
        A = np.zeros((out_size, in_size), np.float64)
        scale = in_size / out_size
        for o in range(out_size):
            src = min(max((o + 0.5) * scale - 0.5, 0.0), in_size - 1)
            i0 = int(np.floor(src))
            i1 = min(i0 + 1, in_size - 1)
            frac = src - i0
            A[o, i0] += 1.0 - frac
            A[o, i1] += frac
        return A
    Ah = axis_weights(H, Hs)
    Aw = axis_weights(W, Ws)
    M = np.einsum("hi,wj->ijhw", Ah, Aw).reshape(Hs * Ws, H * W)
    return M.astype(np.float32)


def _np_tap_masks(H, W):
    p = np.arange(H * W)
    hh, ww = p // W, p % W
    masks = np.zeros((9, H * W), np.float32)
    for k in range(9):
        oy, ox = k // 3 - 1, k % 3 - 1
        valid = ((hh + oy >= 0) & (hh + oy < H)
                 & (ww + ox >= 0) & (ww + ox < W))
        masks[k] = valid.astype(np.float32)
    return masks


def _np_gate_select(TB, C):
    sel = np.zeros((TB * C, TB * C), np.float32)
    rows = np.arange(TB * C)
    sel[rows, (rows // C) * C] = 1.0
    return sel


def _fused_kernel(x_ref, w1_ref, b1_ref, pool_ref, w2_ref, b2_ref, s2_ref,
                  w3_ref, b3_ref, s3_ref, up_ref, gsel_ref, wp_ref, bp_ref,
                  m_ref, o_ref, *, W, HW, halves):
    rows = x_ref.shape[0] // halves
    for h in range(halves):
        _one_group(x_ref, w1_ref, b1_ref, pool_ref, w2_ref, b2_ref, s2_ref,
                   w3_ref, b3_ref, s3_ref, up_ref, gsel_ref, wp_ref, bp_ref,
                   m_ref, o_ref, W=W, HW=HW, r0=h * rows, r1=(h + 1) * rows)


def _one_group(x_ref, w1_ref, b1_ref, pool_ref, w2_ref, b2_ref, s2_ref,
               w3_ref, b3_ref, s3_ref, up_ref, gsel_ref, wp_ref, bp_ref,
               m_ref, o_ref, *, W, HW, r0, r1):
    f32 = jnp.float32
    x = x_ref[r0:r1, :]
    xb = x.astype(_BF)

    x1 = (jnp.dot(w1_ref[...], xb, preferred_element_type=f32)
          + b1_ref[:, 0:1])

    mx = jnp.max(x1, axis=1, keepdims=True)
    e = jnp.exp(x1 - mx)
    num = jnp.dot((e * x1).astype(_BF), pool_ref[...],
                  preferred_element_type=f32)
    den = jnp.dot(e.astype(_BF), pool_ref[...], preferred_element_type=f32)
    x2 = (num / den).astype(_BF)

    n2 = s2_ref.shape[1] // 9
    g2all = jnp.dot(x2, s2_ref[...], preferred_element_type=f32).astype(_BF)
    gstack = jnp.concatenate(
        [g2all[:, k * n2:(k + 1) * n2] for k in range(9)], axis=0)
    a2 = (jnp.dot(w2_ref[...], gstack, preferred_element_type=f32)
          + b2_ref[:, 0:1])

    a2 = a2.astype(_BF)
    g3all = jnp.dot(a2, s3_ref[...], preferred_element_type=f32).astype(_BF)
    g3stack = jnp.concatenate(
        [g3all[:, k * n2:(k + 1) * n2] for k in range(9)], axis=0)
    a3 = (jnp.dot(w3_ref[...], g3stack, preferred_element_type=f32)
          + b3_ref[:, 0:1])
    ys = jax.nn.sigmoid(a3).astype(_BF)

    y = jnp.dot(ys, up_ref[...], preferred_element_type=f32)
    z = x * y
    zb = z.astype(_BF)
    g2 = jax.nn.sigmoid(jnp.dot(gsel_ref[...], xb, preferred_element_type=f32))

    taps = []
    for k in range(9):
        if k == 4:
            taps.append(zb)
            continue
        s = (k // 3 - 1) * W + (k % 3 - 1)
        taps.append(pltpu.roll(zb, shift=(-s) % HW, axis=1) * m_ref[k:k + 1, :])
    zstack = jnp.concatenate(taps, axis=0)
    acc = jnp.dot(wp_ref[...], zstack, preferred_element_type=f32)

    o_ref[r0:r1, :] = z * g2 + acc + bp_ref[:, 0:1]


def _forward(x, params):
    orig_shape = x.shape
    if x.ndim == 5:
        n, s, c, h, w = x.shape
        x4d = x.reshape(n * s, c, h, w)
    elif x.ndim == 4:
        x4d = x
    else:
        raise ValueError("Input tensor must be 4D or 5D")
    B, C, H, W = x4d.shape
    HW = H * W
    f = params["b1"].shape[0]

    TB = 1
    for cand in (16, 8, 4, 2):
        if B % cand == 0:
            TB = cand
            break
    halves = 1
    for cand in (4, 2):
        if B % (cand * TB) == 0:
            halves = cand
            break
    grid = B // (TB * halves)

    Hp, Wp = _out_size(H, 7, 3, 1), _out_size(W, 7, 3, 1)
    H2, W2 = _out_size(Hp, 3, 2, 1), _out_size(Wp, 3, 2, 1)
    n_pool, n2 = Hp * Wp, H2 * W2

    x2d = x4d.reshape(B * C, HW)

    eye = jnp.eye(TB, dtype=jnp.float32)
    w1_bd = jnp.einsum("bd,oc->bodc", eye,
                       params["w1_mat"]).reshape(TB * f, TB * C).astype(_BF)
    w2_cat = jnp.einsum("bd,oiyx->boyxdi", eye,
                        params["w2_oihw"]).reshape(TB * f, 9 * TB * f).astype(_BF)
    w3_cat = jnp.einsum("bd,oiyx->boyxdi", eye,
                        params["w3_oihw"]).reshape(TB * C, 9 * TB * f).astype(_BF)
    wp_cat = jnp.einsum("bd,koi->bokdi", eye,
                        params["wp_taps"]).reshape(TB * C, 9 * TB * C).astype(_BF)
    b1_t = jnp.broadcast_to(jnp.tile(params["b1"], TB)[:, None], (TB * f, 128))
    b2_t = jnp.broadcast_to(jnp.tile(params["b2"], TB)[:, None], (TB * f, 128))
    b3_t = jnp.broadcast_to(jnp.tile(params["b3"], TB)[:, None], (TB * C, 128))
    bp_t = jnp.broadcast_to(jnp.tile(params["bp"], TB)[:, None], (TB * C, 128))

    pool_mat = jnp.asarray(_np_pool_matrix(H, W, 7, 3, 1), dtype=_BF)
    s2_np = _np_conv_gather(Hp, Wp, H2, W2, 2)
    s2 = jnp.asarray(np.transpose(s2_np, (1, 0, 2)).reshape(n_pool, 9 * n2),
                     dtype=_BF)
    s3_np = _np_conv_gather(H2, W2, H2, W2, 1)
    s3 = jnp.asarray(np.transpose(s3_np, (1, 0, 2)).reshape(n2, 9 * n2),
                     dtype=_BF)
    up_mat = jnp.asarray(_np_bilinear_matrix(H2, W2, H, W), dtype=_BF)
    masks = jnp.asarray(_np_tap_masks(H, W), dtype=_BF)
    gsel = jnp.asarray(_np_gate_select(TB, C), dtype=_BF)

    body = functools.partial(_fused_kernel, W=W, HW=HW, halves=halves)
    out2d = pl.pallas_call(
        body,
        out_shape=jax.ShapeDtypeStruct((B * C, HW), jnp.float32),
        grid=(grid,),
        in_specs=[
            pl.BlockSpec((halves * TB * C, HW), lambda i: (i, 0)),
            pl.BlockSpec((TB * f, TB * C), lambda i: (0, 0)),
            pl.BlockSpec((TB * f, 128), lambda i: (0, 0)),
            pl.BlockSpec((HW, n_pool), lambda i: (0, 0)),
            pl.BlockSpec((TB * f, 9 * TB * f), lambda i: (0, 0)),
            pl.BlockSpec((TB * f, 128), lambda i: (0, 0)),
            pl.BlockSpec((n_pool, 9 * n2), lambda i: (0, 0)),
            pl.BlockSpec((TB * C, 9 * TB * f), lambda i: (0, 0)),
            pl.BlockSpec((TB * C, 128), lambda i: (0, 0)),
            pl.BlockSpec((n2, 9 * n2), lambda i: (0, 0)),
            pl.BlockSpec((n2, HW), lambda i: (0, 0)),
            pl.BlockSpec((TB * C, TB * C), lambda i: (0, 0)),
            pl.BlockSpec((TB * C, 9 * TB * C), lambda i: (0, 0)),
            pl.BlockSpec((TB * C, 128), lambda i: (0, 0)),
            pl.BlockSpec((9, HW), lambda i: (0, 0)),
        ],
        out_specs=pl.BlockSpec((halves * TB * C, HW), lambda i: (i, 0)),
        compiler_params=pltpu.CompilerParams(
            dimension_semantics=("parallel",),
            allow_input_fusion=[False] + [True] * 14,
            vmem_limit_bytes=64 << 20),
        cost_estimate=pl.CostEstimate(
            flops=2 * B * HW * (C * f + 2 * f * n_pool // 5 + 10 * C * C)
            + 2 * B * f * 9 * (f * n2 + n_pool * n2 // 4),
            transcendentals=B * (f + C) * HW,
            bytes_accessed=4 * 2 * B * C * HW),
    )(x2d, w1_bd, b1_t, pool_mat, w2_cat, b2_t, s2, w3_cat, b3_t, s3,
      up_mat, gsel, wp_cat, bp_t, masks)

    out = out2d.reshape(B, C, H, W)
    return out.reshape(orig_shape)


def kernel(x, w1_oihw, b1, w1_mat, w2_oihw, b2, w3_oihw, b3, wp_oihw, bp,
           wp_taps):
    params = dict(
        w1_mat=w1_mat, b1=b1,
        w2_oihw=w2_oihw, b2=b2,
        w3_oihw=w3_oihw, b3=b3,
        wp_taps=wp_taps, bp=bp,
    )
    return _forward(x, params)


# speedup vs baseline: 69.3538x; 1.0096x over previous
import functools
import math

import numpy as np
import jax
import jax.numpy as jnp
from jax import lax
from jax.experimental import pallas as pl
from jax.experimental.pallas import tpu as pltpu

_BF = jnp.bfloat16


def _out_size(n, k, s, p):
    return (n + 2 * p - k) // s + 1


def _np_pool_matrix(H, W, k, s, p):
    Ho, Wo = _out_size(H, k, s, p), _out_size(W, k, s, p)
    P = np.zeros((H * W, Ho * Wo), np.float32)
    for qy in range(Ho):
        for qx in range(Wo):
            y0, x0 = qy * s - p, qx * s - p
            for dy in range(k):
                for dx in range(k):
                    yy, xx = y0 + dy, x0 + dx
                    if 0 <= yy < H and 0 <= xx < W:
                        P[yy * W + xx, qy * Wo + qx] = 1.0
    return P


def _np_conv_gather_cat(Hi, Wi, Ho, Wo, stride):
    S = np.zeros((Hi * Wi, 9 * Ho * Wo), np.float32)
    for kk in range(9):
        dy, dx = kk // 3 - 1, kk % 3 - 1
        for qy in range(Ho):
            for qx in range(Wo):
                yy, xx = qy * stride + dy, qx * stride + dx
                if 0 <= yy < Hi and 0 <= xx < Wi:
                    S[yy * Wi + xx, kk * Ho * Wo + qy * Wo + qx] = 1.0
    return S


def _np_bilinear_matrix(Hs, Ws, H, W):
    def axis_weights(out_size, in_size):
        A = np.zeros((out_size, in_size), np.float64)
        scale = in_size / out_size
        for o in range(out_size):
            src = min(max((o + 0.5) * scale - 0.5, 0.0), in_size - 1)
            i0 = int(np.floor(src))
            i1 = min(i0 + 1, in_size - 1)
            frac = src - i0
            A[o, i0] += 1.0 - frac
            A[o, i1] += frac
        return A
    Ah = axis_weights(H, Hs)
    Aw = axis_weights(W, Ws)
    M = np.einsum("hi,wj->ijhw", Ah, Aw).reshape(Hs * Ws, H * W)
    return M.astype(np.float32)


def _np_tap_masks(H, W):
    p = np.arange(H * W)
    hh, ww = p // W, p % W
    masks = np.zeros((9, H * W), np.float32)
    for k in range(9):
        oy, ox = k // 3 - 1, k % 3 - 1
        valid = ((hh + oy >= 0) & (hh + oy < H)
                 & (ww + ox >= 0) & (ww + ox < W))
        masks[k] = valid.astype(np.float32)
    return masks


def _np_gate_select(TB, C):
    sel = np.zeros((TB * C, TB * C), np.float32)
    rows = np.arange(TB * C)
    sel[rows, (rows // C) * C] = 1.0
    return sel


def _np_r_cat(f, TB):
    R = np.zeros((f * 9, 9 * TB * f), np.float32)
    for i in range(f):
        for k in range(9):
            for d in range(TB):
                R[i * 9 + k, k * TB * f + d * f + i] = 1.0
    return R


def _fused_kernel(x_ref, w1c_ref, w2f_ref, w3f_ref, wpf_ref,
                  b1c_ref, b2c_ref, b3c_ref, bpc_ref,
                  pool_ref, s2_ref, s3_ref, up_ref, gsel_ref, m_ref,
                  vf_ref, vc_ref, r1_ref, m1bd_ref, r2_ref, m2_ref, m3_ref,
                  e9_ref, m9_ref, u9_ref, mp_ref,
                  o_ref, w1s, w2s, w3s, wps,
                  *, W, HW, halves):
    f32 = jnp.float32

    @pl.when(pl.program_id(0) == 0)
    def _build_weights():
        vf = vf_ref[...]
        vc = vc_ref[...]
        t1 = jnp.dot(vf, w1c_ref[...].astype(_BF),
                     preferred_element_type=f32).astype(_BF)
        w1s[...] = jnp.dot(t1, r1_ref[...],
                           preferred_element_type=f32).astype(_BF) * m1bd_ref[...]
        t2 = jnp.dot(vf, w2f_ref[...].astype(_BF),
                     preferred_element_type=f32).astype(_BF)
        w2s[...] = jnp.dot(t2, r2_ref[...],
                           preferred_element_type=f32).astype(_BF) * m2_ref[...]
        t3 = jnp.dot(vc, w3f_ref[...].astype(_BF),
                     preferred_element_type=f32).astype(_BF)
        w3s[...] = jnp.dot(t3, r2_ref[...],
                           preferred_element_type=f32).astype(_BF) * m3_ref[...]
        xp = (jnp.dot(wpf_ref[...].astype(_BF), e9_ref[...],
                      preferred_element_type=f32).astype(_BF) * m9_ref[...])
        wps[...] = jnp.dot(u9_ref[...], xp,
                           preferred_element_type=f32).astype(_BF) * mp_ref[...]

    rows = x_ref.shape[0] // halves
    for h in range(halves):
        _one_group(x_ref, b1c_ref, b2c_ref, b3c_ref, bpc_ref,
                   pool_ref, s2_ref, s3_ref, up_ref, gsel_ref, m_ref,
                   o_ref, w1s, w2s, w3s, wps,
                   W=W, HW=HW, r0=h * rows, r1=(h + 1) * rows)


def _one_group(x_ref, b1c_ref, b2c_ref, b3c_ref, bpc_ref,
               pool_ref, s2_ref, s3_ref, up_ref, gsel_ref, m_ref,
               o_ref, w1s, w2s, w3s, wps, *, W, HW, r0, r1):
    f32 = jnp.float32
    TB = w1s.shape[0] // b1c_ref.shape[0]
    x = x_ref[r0:r1, :]
    xb = x.astype(_BF)

    b1t = jnp.tile(b1c_ref[...], (TB, 1))
    x1 = jnp.dot(w1s[...], xb, preferred_element_type=f32) + b1t

    mx = jnp.max(x1, axis=1, keepdims=True)
    e = jnp.exp(x1 - mx)
    num = jnp.dot((e * x1).astype(_BF), pool_ref[...],
                  preferred_element_type=f32)
    den = jnp.dot(e.astype(_BF), pool_ref[...], preferred_element_type=f32)
    x2 = (num / den).astype(_BF)

    n2 = s2_ref.shape[1] // 9
    g2all = jnp.dot(x2, s2_ref[...], preferred_element_type=f32).astype(_BF)
    gstack = jnp.concatenate(
        [g2all[:, k * n2:(k + 1) * n2] for k in range(9)], axis=0)
    a2 = (jnp.dot(w2s[...], gstack, preferred_element_type=f32)
          + jnp.tile(b2c_ref[...], (TB, 1)))

    a2 = a2.astype(_BF)
    g3all = jnp.dot(a2, s3_ref[...], preferred_element_type=f32).astype(_BF)
    g3stack = jnp.concatenate(
        [g3all[:, k * n2:(k + 1) * n2] for k in range(9)], axis=0)
    a3 = (jnp.dot(w3s[...], g3stack, preferred_element_type=f32)
          + jnp.tile(b3c_ref[...], (TB, 1)))
    ys = jax.nn.sigmoid(a3).astype(_BF)

    y = jnp.dot(ys, up_ref[...], preferred_element_type=f32)
    z = x * y
    zb = z.astype(_BF)
    g2 = jax.nn.sigmoid(jnp.dot(gsel_ref[...], xb, preferred_element_type=f32))

    taps = []
    for k in range(9):
        if k == 4:
            taps.append(zb)
            continue
        s = (k // 3 - 1) * W + (k % 3 - 1)
        taps.append(pltpu.roll(zb, shift=(-s) % HW, axis=1) * m_ref[k:k + 1, :])
    zstack = jnp.concatenate(taps, axis=0)
    acc = jnp.dot(wps[...], zstack, preferred_element_type=f32)

    o_ref[r0:r1, :] = z * g2 + acc + jnp.tile(bpc_ref[...], (TB, 1))


def _forward(x, params):
    orig_shape = x.shape
    if x.ndim == 5:
        n, s, c, h, w = x.shape
        x4d = x.reshape(n * s, c, h, w)
    elif x.ndim == 4:
        x4d = x
    else:
        raise ValueError("Input tensor must be 4D or 5D")
    B, C, H, W = x4d.shape
    HW = H * W
    f = params["b1"].shape[0]

    TB = 1
    for cand in (16, 8, 4, 2):
        if B % cand == 0:
            TB = cand
            break
    halves = 1
    for cand in (4, 2):
        if B % (cand * TB) == 0:
            halves = cand
            break
    grid = B // (TB * halves)

    Hp, Wp = _out_size(H, 7, 3, 1), _out_size(W, 7, 3, 1)
    H2, W2 = _out_size(Hp, 3, 2, 1), _out_size(Wp, 3, 2, 1)
    n_pool, n2 = Hp * Wp, H2 * W2

    x2d = x4d.reshape(B * C, HW)

    w1c = params["w1_mat"]
    w2f = params["w2_oihw"].reshape(f, 9 * f)
    w3f = params["w3_oihw"].reshape(C, 9 * f)
    wpf = params["wp_taps"].reshape(9 * C, C)
    b1c = params["b1"].reshape(f, 1)
    b2c = params["b2"].reshape(f, 1)
    b3c = params["b3"].reshape(C, 1)
    bpc = params["bp"].reshape(C, 1)

    pool_mat = jnp.asarray(_np_pool_matrix(H, W, 7, 3, 1), dtype=_BF)
    s2 = jnp.asarray(_np_conv_gather_cat(Hp, Wp, H2, W2, 2), dtype=_BF)
    s3 = jnp.asarray(_np_conv_gather_cat(H2, W2, H2, W2, 1), dtype=_BF)
    up_mat = jnp.asarray(_np_bilinear_matrix(H2, W2, H, W), dtype=_BF)
    masks = jnp.asarray(_np_tap_masks(H, W), dtype=_BF)
    gsel = jnp.asarray(_np_gate_select(TB, C), dtype=_BF)
    vf = jnp.asarray(np.tile(np.eye(f, dtype=np.float32), (TB, 1)), dtype=_BF)
    vc = jnp.asarray(np.tile(np.eye(C, dtype=np.float32), (TB, 1)), dtype=_BF)
    r1 = jnp.asarray(np.tile(np.eye(C, dtype=np.float32), (1, TB)), dtype=_BF)
    m1bd = jnp.asarray(np.kron(np.eye(TB), np.ones((f, C))), dtype=_BF)
    r2 = jnp.asarray(_np_r_cat(f, TB), dtype=_BF)
    m2 = jnp.asarray(np.tile(np.kron(np.eye(TB), np.ones((f, f))), (1, 9)),
                     dtype=_BF)
    m3 = jnp.asarray(np.tile(np.kron(np.eye(TB), np.ones((C, f))), (1, 9)),
                     dtype=_BF)
    e9 = jnp.asarray(np.tile(np.eye(C, dtype=np.float32), (1, 9 * TB)),
                     dtype=_BF)
    m9 = jnp.asarray(np.kron(np.eye(9), np.ones((C, TB * C))), dtype=_BF)
    u9 = jnp.asarray(np.tile(np.eye(C, dtype=np.float32), (TB, 9)), dtype=_BF)
    mp = jnp.asarray(np.tile(np.kron(np.eye(TB), np.ones((C, C))), (1, 9)),
                     dtype=_BF)

    def whole(a):
        return pl.BlockSpec(a.shape, lambda i: (0,) * a.ndim)

    body = functools.partial(_fused_kernel, W=W, HW=HW, halves=halves)
    out2d = pl.pallas_call(
        body,
        out_shape=jax.ShapeDtypeStruct((B * C, HW), jnp.float32),
        grid=(grid,),
        in_specs=[
            pl.BlockSpec((halves * TB * C, HW), lambda i: (i, 0)),
            whole(w1c), whole(w2f), whole(w3f), whole(wpf),
            whole(b1c), whole(b2c), whole(b3c), whole(bpc),
            whole(pool_mat), whole(s2), whole(s3), whole(up_mat),
            whole(gsel), whole(masks),
            whole(vf), whole(vc), whole(r1), whole(m1bd), whole(r2),
            whole(m2), whole(m3), whole(e9), whole(m9), whole(u9), whole(mp),
        ],
        out_specs=pl.BlockSpec((halves * TB * C, HW), lambda i: (i, 0)),
        scratch_shapes=[
            pltpu.VMEM((TB * f, TB * C), _BF),
            pltpu.VMEM((TB * f, 9 * TB * f), _BF),
            pltpu.VMEM((TB * C, 9 * TB * f), _BF),
            pltpu.VMEM((TB * C, 9 * TB * C), _BF),
        ],
        compiler_params=pltpu.CompilerParams(
            dimension_semantics=("arbitrary",),
            vmem_limit_bytes=60000 * 1024),
        cost_estimate=pl.CostEstimate(
            flops=2 * B * HW * (C * f + 2 * f * n_pool // 5 + 10 * C * C)
            + 2 * B * f * 9 * (f * n2 + n_pool * n2 // 4),
            transcendentals=B * (f + C) * HW,
            bytes_accessed=4 * 2 * B * C * HW),
    )(x2d, w1c, w2f, w3f, wpf, b1c, b2c, b3c, bpc,
      pool_mat, s2, s3, up_mat, gsel, masks,
      vf, vc, r1, m1bd, r2, m2, m3, e9, m9, u9, mp)

    out = out2d.reshape(B, C, H, W)
    return out.reshape(orig_shape)


def kernel(x, w1_oihw, b1, w1_mat, w2_oihw, b2, w3_oihw, b3, wp_oihw, bp,
           wp_taps):
    params = dict(
        w1_mat=w1_mat, b1=b1,
        w2_oihw=w2_oihw, b2=b2,
        w3_oihw=w3_oihw, b3=b3,
        wp_taps=wp_taps, bp=bp,
    )
    return _forward(x, params)
